# baseline (speedup 1.0000x reference)
import functools

import jax
import jax.numpy as jnp
from jax import lax
from jax.experimental import pallas as pl
from jax.experimental.pallas import tpu as pltpu
from jax.experimental.pallas import tpu_sc as plsc

NC = 2
NS = 16
NW = NC * NS
CH = 80


def _silu(v):
    return v * jax.nn.sigmoid(v)


def _prep_body(h_ref, g_ref, b_ref, wa_ref, wb_ref, hb_ref, a_ref, bb_ref):
    h = h_ref[...]
    mean = jnp.mean(h, axis=0, keepdims=True)
    var = jnp.mean((h - mean) ** 2, axis=0, keepdims=True)
    hb = g_ref[...] * (h - mean) / jnp.sqrt(var + 1e-5) + b_ref[...]
    hb_ref[...] = hb
    a_ref[...] = jnp.dot(hb, wa_ref[...],
                         preferred_element_type=jnp.float32).astype(jnp.bfloat16)
    bb_ref[...] = jnp.dot(hb, wb_ref[...],
                          preferred_element_type=jnp.float32).astype(jnp.bfloat16)


def _row_add(dst, src, width):
    def row(r, carry):
        for g in range(width // 32):
            sl = pl.ds(g * 32, 32)
            dst[r, sl] = dst[r, sl] + src[r, sl]
        return carry

    lax.fori_loop(0, CH, row, 0)


def _gather_body(n_nodes, epw, a_hbm, b_hbm, xp_hbm, es_hbm, ed_hbm,
                 p0_out, xs_out, xd_out,
                 ids_s, ids_d, asb0, bdb0, xsb0, xdb0, asb1, bdb1, xsb1, xdb1,
                 sem_i, sem_g0, sem_g1, sem_o0, sem_o1):
    c = lax.axis_index("c")
    s = lax.axis_index("s")
    base = (s * NC + c) * epw
    cp1 = pltpu.async_copy(es_hbm.at[pl.ds(base, epw)], ids_s, sem_i)
    cp2 = pltpu.async_copy(ed_hbm.at[pl.ds(base, epw)], ids_d, sem_i)
    cp1.wait()
    cp2.wait()

    sets = ((asb0, bdb0, xsb0, xdb0, sem_g0, sem_o0),
            (asb1, bdb1, xsb1, xdb1, sem_g1, sem_o1))

    def fire(j, st):
        asb, bdb, xsb, xdb, sem_g, _ = st
        loc = j * CH
        si = ids_s.at[pl.ds(loc, CH)]
        di = ids_d.at[pl.ds(loc, CH)]
        return (pltpu.async_copy(a_hbm.at[si], asb, sem_g),
                pltpu.async_copy(b_hbm.at[di], bdb, sem_g),
                pltpu.async_copy(xp_hbm.at[si], xsb, sem_g),
                pltpu.async_copy(xp_hbm.at[di], xdb, sem_g))

    def finish(j, st, cps):
        asb, bdb, xsb, xdb, _, sem_o = st
        for cp in cps:
            cp.wait()
        _row_add(asb, bdb, asb.shape[1])
        off = base + j * CH
        return (pltpu.async_copy(asb, p0_out.at[pl.ds(off, CH)], sem_o),
                pltpu.async_copy(xsb, xs_out.at[pl.ds(off, CH)], sem_o),
                pltpu.async_copy(xdb, xd_out.at[pl.ds(off, CH)], sem_o))

    nch = epw // CH

    def pair(jj, carry):
        j0 = jj * 2
        cps0 = fire(j0, sets[0])
        cps1 = fire(j0 + 1, sets[1])
        out0 = finish(j0, sets[0], cps0)
        out1 = finish(j0 + 1, sets[1], cps1)
        for cp in out0 + out1:
            cp.wait()
        return carry

    lax.fori_loop(0, nch // 2, pair, 0)
    if nch % 2:
        j = nch - 1
        cps = fire(j, sets[0])
        outs = finish(j, sets[0], cps)
        for cp in outs:
            cp.wait()


def _edge_body(p0_ref, xs_ref, xd_ref, w256_ref, be1_ref, we2_ref,
               be2_ref, wit_ref, bi_ref, out_ref):
    diff = xs_ref[...] - xd_ref[...]
    dist = jnp.sqrt(jnp.sum(diff * diff, axis=1, keepdims=True))
    pre1 = p0_ref[...].astype(jnp.float32) + dist * w256_ref[...] + be1_ref[...]
    u = _silu(pre1).astype(jnp.bfloat16)
    v = jnp.dot(u, we2_ref[...], preferred_element_type=jnp.float32)
    v = _silu(v + be2_ref[...])
    logit = jnp.sum(v * wit_ref[...], axis=1, keepdims=True) + bi_ref[0]
    out_ref[...] = jax.nn.sigmoid(logit) * v


def _scatter_body(n_nodes, epw, wm_hbm, es_hbm, zeros_hbm, out_hbm,
                  acc, rows0, idx0, rows1, idx1,
                  sem_l0, sem_s0, sem_l1, sem_s1):
    c = lax.axis_index("c")
    s = lax.axis_index("s")
    npt = (n_nodes // NS) // 8 * 8
    rem = n_nodes - npt * NS
    pltpu.sync_copy(zeros_hbm.at[pl.ds(s * npt, npt)],
                    acc.at[pl.ds(s * npt, npt)])

    @pl.when(s == 0)
    def _():
        pltpu.sync_copy(zeros_hbm.at[pl.ds(npt * NS, rem)],
                        acc.at[pl.ds(npt * NS, rem)])

    plsc.subcore_barrier()
    base = (s * NC + c) * epw

    sets = ((rows0, idx0, sem_l0, sem_s0), (rows1, idx1, sem_l1, sem_s1))

    def load(j, st):
        rows, idx, sem_l, _ = st
        off = base + j * CH
        return (pltpu.async_copy(wm_hbm.at[pl.ds(off, CH)], rows, sem_l),
                pltpu.async_copy(es_hbm.at[pl.ds(off, CH)], idx, sem_l))

    def scat(st, cps):
        rows, idx, _, sem_s = st
        for cp in cps:
            cp.wait()
        return pltpu.async_copy(rows, acc.at[idx], sem_s, add=True)

    nch = epw // CH

    def pair(jj, carry):
        j0 = jj * 2
        cps0 = load(j0, sets[0])
        cps1 = load(j0 + 1, sets[1])
        sc0 = scat(sets[0], cps0)
        sc1 = scat(sets[1], cps1)
        sc0.wait()
        sc1.wait()
        return carry

    lax.fori_loop(0, nch // 2, pair, 0)
    if nch % 2:
        cps = load(nch - 1, sets[0])
        scat(sets[0], cps).wait()
    plsc.subcore_barrier()
    obase = c * n_nodes + s * npt
    pltpu.sync_copy(acc.at[pl.ds(s * npt, npt)], out_hbm.at[pl.ds(obase, npt)])

    @pl.when(s == 0)
    def _():
        pltpu.sync_copy(acc.at[pl.ds(npt * NS, rem)],
                        out_hbm.at[pl.ds(c * n_nodes + npt * NS, rem)])


def _final_body(n_nodes, hb_ref, mp_ref, wh1h_ref, wh1m_ref, bh1_ref,
                wh2_ref, bh2_ref, out_ref):
    hb = hb_ref[...]
    m = mp_ref[:n_nodes, :] + mp_ref[n_nodes:, :]
    z = (jnp.dot(hb, wh1h_ref[...], preferred_element_type=jnp.float32)
         + jnp.dot(m, wh1m_ref[...], preferred_element_type=jnp.float32)
         + bh1_ref[...])
    z = _silu(z)
    out_ref[...] = hb + jnp.dot(z, wh2_ref[...],
                                preferred_element_type=jnp.float32) + bh2_ref[...]


def kernel(h, x, e, gamma, beta, We1, be1, We2, be2, Wi, bi, Wh1, bh1, Wh2, bh2):
    n, d = h.shape
    ne = e.shape[1]
    hh = We2.shape[0]
    epw = ne // NW
    mesh = plsc.VectorSubcoreMesh(core_axis_name="c", subcore_axis_name="s")

    hb, A, B = pl.pallas_call(
        _prep_body,
        out_shape=[jax.ShapeDtypeStruct((n, d), jnp.float32),
                   jax.ShapeDtypeStruct((n, hh), jnp.bfloat16),
                   jax.ShapeDtypeStruct((n, hh), jnp.bfloat16)],
    )(h, gamma.reshape(1, d), beta.reshape(1, d), We1[:d], We1[d:2 * d])

    es = e[0]
    ed = e[1]
    xp = jnp.pad(x.astype(jnp.float32), ((0, 0), (0, 16 - x.shape[1])))
    gather = pl.kernel(
        functools.partial(_gather_body, n, epw),
        out_type=[jax.ShapeDtypeStruct((ne, hh), jnp.bfloat16),
                  jax.ShapeDtypeStruct((ne, 16), jnp.float32),
                  jax.ShapeDtypeStruct((ne, 16), jnp.float32)],
        mesh=mesh,
        scratch_types=[pltpu.VMEM((epw,), jnp.int32),
                       pltpu.VMEM((epw,), jnp.int32),
                       pltpu.VMEM((CH, hh), jnp.bfloat16),
                       pltpu.VMEM((CH, hh), jnp.bfloat16),
                       pltpu.VMEM((CH, 16), jnp.float32),
                       pltpu.VMEM((CH, 16), jnp.float32),
                       pltpu.VMEM((CH, hh), jnp.bfloat16),
                       pltpu.VMEM((CH, hh), jnp.bfloat16),
                       pltpu.VMEM((CH, 16), jnp.float32),
                       pltpu.VMEM((CH, 16), jnp.float32),
                       pltpu.SemaphoreType.DMA,
                       pltpu.SemaphoreType.DMA,
                       pltpu.SemaphoreType.DMA,
                       pltpu.SemaphoreType.DMA,
                       pltpu.SemaphoreType.DMA],
        compiler_params=pltpu.CompilerParams(use_tc_tiling_on_sc=False),
    )
    P0, Xs, Xd = gather(A, B, xp, es, ed)

    R = 2000
    grid = ne // R
    wm = pl.pallas_call(
        _edge_body,
        grid=(grid,),
        in_specs=[
            pl.BlockSpec((R, hh), lambda i: (i, 0)),
            pl.BlockSpec((R, 16), lambda i: (i, 0)),
            pl.BlockSpec((R, 16), lambda i: (i, 0)),
            pl.BlockSpec((1, hh), lambda i: (0, 0)),
            pl.BlockSpec((1, hh), lambda i: (0, 0)),
            pl.BlockSpec((hh, hh), lambda i: (0, 0)),
            pl.BlockSpec((1, hh), lambda i: (0, 0)),
            pl.BlockSpec((1, hh), lambda i: (0, 0)),
            pl.BlockSpec(memory_space=pltpu.MemorySpace.SMEM),
        ],
        out_specs=pl.BlockSpec((R, hh), lambda i: (i, 0)),
        out_shape=jax.ShapeDtypeStruct((ne, hh), jnp.float32),
    )(P0, Xs, Xd, We1[2 * d:2 * d + 1], be1.reshape(1, hh),
      We2.astype(jnp.bfloat16), be2.reshape(1, hh), Wi.reshape(1, hh), bi)

    zeros = jnp.zeros((n, hh), jnp.float32)
    scatter = pl.kernel(
        functools.partial(_scatter_body, n, epw),
        out_type=jax.ShapeDtypeStruct((NC * n, hh), jnp.float32),
        mesh=mesh,
        scratch_types=[pltpu.VMEM_SHARED((n, hh), jnp.float32),
                       pltpu.VMEM((CH, hh), jnp.float32),
                       pltpu.VMEM((CH,), jnp.int32),
                       pltpu.VMEM((CH, hh), jnp.float32),
                       pltpu.VMEM((CH,), jnp.int32),
                       pltpu.SemaphoreType.DMA,
                       pltpu.SemaphoreType.DMA,
                       pltpu.SemaphoreType.DMA,
                       pltpu.SemaphoreType.DMA],
    )
    mparts = scatter(wm, es, zeros)

    h_out = pl.pallas_call(
        functools.partial(_final_body, n),
        out_shape=jax.ShapeDtypeStruct((n, d), jnp.float32),
    )(hb, mparts, Wh1[:d], Wh1[d:], bh1.reshape(1, hh), Wh2,
      bh2.reshape(1, d))

    return (h_out, e)


# speedup vs baseline: 1.0981x; 1.0981x over previous
import functools

import jax
import jax.numpy as jnp
from jax import lax
from jax.experimental import pallas as pl
from jax.experimental.pallas import tpu as pltpu
from jax.experimental.pallas import tpu_sc as plsc

NC = 2
NS = 16
NW = NC * NS
CH = 80


def _silu(v):
    return v * jax.nn.sigmoid(v)


def _prep_body(h_ref, g_ref, b_ref, wa_ref, wb_ref, be1_ref,
               hb_ref, a_ref, bb_ref):
    h = h_ref[...]
    mean = jnp.mean(h, axis=0, keepdims=True)
    var = jnp.mean((h - mean) ** 2, axis=0, keepdims=True)
    hb = g_ref[...] * (h - mean) / jnp.sqrt(var + 1e-5) + b_ref[...]
    hb_ref[...] = hb
    half_be1 = 0.5 * be1_ref[...]
    a_ref[...] = jnp.dot(hb, wa_ref[...],
                         preferred_element_type=jnp.float32) + half_be1
    bb_ref[...] = jnp.dot(hb, wb_ref[...],
                          preferred_element_type=jnp.float32) + half_be1


def _chunk_compute(asb, bdb, x0s, x1s, x2s, x0d, x1d, x2d, d2b, wv, width):
    rs2 = 0.7071067811865476

    def blkd(i, carry):
        sl = pl.ds(i * 16, 16)
        dx = x0s[sl] - x0d[sl]
        dy = x1s[sl] - x1d[sl]
        dz = x2s[sl] - x2d[sl]
        d2 = dx * dx + dy * dy + dz * dz
        y = jnp.full((16,), rs2 * 2.0 ** 8, jnp.float32)
        for k in range(-7, 5):
            y = jnp.where(d2 >= 4.0 ** k,
                          jnp.float32(rs2 * 2.0 ** (-k)), y)
        for _ in range(5):
            y = y * (1.5 - 0.5 * d2 * y * y)
        d2b[sl] = d2 * y
        return carry

    lax.fori_loop(0, CH // 16, blkd, 0)

    def blk3(i, carry):
        dist16 = d2b[pl.ds(i * 16, 16)]
        for m in range(16):
            r = i * 16 + m
            dist = dist16[m]
            for g in range(width // 16):
                sl = pl.ds(g * 16, 16)
                asb[r, sl] = asb[r, sl] + bdb[r, sl] + dist * wv[sl]
        return carry

    lax.fori_loop(0, CH // 16, blk3, 0)


def _gather_body(n_nodes, epw, a_hbm, b_hbm, x0_hbm, x1_hbm, x2_hbm, w_hbm,
                 es_hbm, ed_hbm, p1_out,
                 ids_s, ids_d, wv, d2b,
                 asb0, bdb0, x0s0, x1s0, x2s0, x0d0, x1d0, x2d0,
                 asb1, bdb1, x0s1, x1s1, x2s1, x0d1, x1d1, x2d1,
                 sem_i, sem_g0, sem_g1, sem_o0, sem_o1):
    c = lax.axis_index("c")
    s = lax.axis_index("s")
    base = (s * NC + c) * epw
    cp1 = pltpu.async_copy(es_hbm.at[pl.ds(base, epw)], ids_s, sem_i)
    cp2 = pltpu.async_copy(ed_hbm.at[pl.ds(base, epw)], ids_d, sem_i)
    pltpu.sync_copy(w_hbm, wv)
    cp1.wait()
    cp2.wait()

    sets = ((asb0, bdb0, x0s0, x1s0, x2s0, x0d0, x1d0, x2d0, sem_g0, sem_o0),
            (asb1, bdb1, x0s1, x1s1, x2s1, x0d1, x1d1, x2d1, sem_g1, sem_o1))

    def fire(j, st):
        asb, bdb, x0s, x1s, x2s, x0d, x1d, x2d, sem_g, _ = st
        loc = j * CH
        si = ids_s.at[pl.ds(loc, CH)]
        di = ids_d.at[pl.ds(loc, CH)]
        return (pltpu.async_copy(a_hbm.at[si], asb, sem_g),
                pltpu.async_copy(b_hbm.at[di], bdb, sem_g),
                pltpu.async_copy(x0_hbm.at[si], x0s, sem_g),
                pltpu.async_copy(x1_hbm.at[si], x1s, sem_g),
                pltpu.async_copy(x2_hbm.at[si], x2s, sem_g),
                pltpu.async_copy(x0_hbm.at[di], x0d, sem_g),
                pltpu.async_copy(x1_hbm.at[di], x1d, sem_g),
                pltpu.async_copy(x2_hbm.at[di], x2d, sem_g))

    def finish(j, st, cps):
        asb, bdb, x0s, x1s, x2s, x0d, x1d, x2d, _, sem_o = st
        for cp in cps:
            cp.wait()
        _chunk_compute(asb, bdb, x0s, x1s, x2s, x0d, x1d, x2d, d2b, wv,
                       asb.shape[1])
        off = base + j * CH
        return (pltpu.async_copy(asb, p1_out.at[pl.ds(off, CH)], sem_o),)

    nch = epw // CH

    def pair(jj, carry):
        j0 = jj * 2
        cps0 = fire(j0, sets[0])
        cps1 = fire(j0 + 1, sets[1])
        out0 = finish(j0, sets[0], cps0)
        out1 = finish(j0 + 1, sets[1], cps1)
        for cp in out0 + out1:
            cp.wait()
        return carry

    lax.fori_loop(0, nch // 2, pair, 0)
    if nch % 2:
        j = nch - 1
        cps = fire(j, sets[0])
        outs = finish(j, sets[0], cps)
        for cp in outs:
            cp.wait()


def _edge_body(p1_ref, we2_ref, be2_ref, wit_ref, bi_ref, out_ref):
    u = _silu(p1_ref[...])
    v = jnp.dot(u, we2_ref[...], preferred_element_type=jnp.float32)
    v = _silu(v + be2_ref[...])
    logit = jnp.sum(v * wit_ref[...], axis=1, keepdims=True) + bi_ref[0]
    out_ref[...] = jax.nn.sigmoid(logit) * v


def _scatter_body(n_nodes, epw, wm_hbm, es_hbm, zeros_hbm, out_hbm,
                  acc, rows0, idx0, rows1, idx1,
                  sem_l0, sem_s0, sem_l1, sem_s1):
    c = lax.axis_index("c")
    s = lax.axis_index("s")
    npt = (n_nodes // NS) // 8 * 8
    rem = n_nodes - npt * NS
    pltpu.sync_copy(zeros_hbm.at[pl.ds(s * npt, npt)],
                    acc.at[pl.ds(s * npt, npt)])

    @pl.when(s == 0)
    def _():
        pltpu.sync_copy(zeros_hbm.at[pl.ds(npt * NS, rem)],
                        acc.at[pl.ds(npt * NS, rem)])

    plsc.subcore_barrier()
    base = (s * NC + c) * epw

    sets = ((rows0, idx0, sem_l0, sem_s0), (rows1, idx1, sem_l1, sem_s1))

    def load(j, st):
        rows, idx, sem_l, _ = st
        off = base + j * CH
        return (pltpu.async_copy(wm_hbm.at[pl.ds(off, CH)], rows, sem_l),
                pltpu.async_copy(es_hbm.at[pl.ds(off, CH)], idx, sem_l))

    def scat(st, cps):
        rows, idx, _, sem_s = st
        for cp in cps:
            cp.wait()
        return pltpu.async_copy(rows, acc.at[idx], sem_s, add=True)

    nch = epw // CH

    def pair(jj, carry):
        j0 = jj * 2
        cps0 = load(j0, sets[0])
        cps1 = load(j0 + 1, sets[1])
        sc0 = scat(sets[0], cps0)
        sc1 = scat(sets[1], cps1)
        sc0.wait()
        sc1.wait()
        return carry

    lax.fori_loop(0, nch // 2, pair, 0)
    if nch % 2:
        cps = load(nch - 1, sets[0])
        scat(sets[0], cps).wait()
    plsc.subcore_barrier()
    obase = c * n_nodes + s * npt
    pltpu.sync_copy(acc.at[pl.ds(s * npt, npt)], out_hbm.at[pl.ds(obase, npt)])

    @pl.when(s == 0)
    def _():
        pltpu.sync_copy(acc.at[pl.ds(npt * NS, rem)],
                        out_hbm.at[pl.ds(c * n_nodes + npt * NS, rem)])


def _final_body(n_nodes, hb_ref, mp_ref, wh1h_ref, wh1m_ref, bh1_ref,
                wh2_ref, bh2_ref, out_ref):
    hb = hb_ref[...]
    m = mp_ref[:n_nodes, :] + mp_ref[n_nodes:, :]
    z = (jnp.dot(hb, wh1h_ref[...], preferred_element_type=jnp.float32)
         + jnp.dot(m, wh1m_ref[...], preferred_element_type=jnp.float32)
         + bh1_ref[...])
    z = _silu(z)
    out_ref[...] = hb + jnp.dot(z, wh2_ref[...],
                                preferred_element_type=jnp.float32) + bh2_ref[...]


def kernel(h, x, e, gamma, beta, We1, be1, We2, be2, Wi, bi, Wh1, bh1, Wh2, bh2):
    n, d = h.shape
    ne = e.shape[1]
    hh = We2.shape[0]
    epw = ne // NW
    mesh = plsc.VectorSubcoreMesh(core_axis_name="c", subcore_axis_name="s")

    hb, A, B = pl.pallas_call(
        _prep_body,
        out_shape=[jax.ShapeDtypeStruct((n, d), jnp.float32),
                   jax.ShapeDtypeStruct((n, hh), jnp.float32),
                   jax.ShapeDtypeStruct((n, hh), jnp.float32)],
    )(h, gamma.reshape(1, d), beta.reshape(1, d), We1[:d], We1[d:2 * d],
      be1.reshape(1, hh))

    es = e[0]
    ed = e[1]
    x0a = x[:, 0]
    x1a = x[:, 1]
    x2a = x[:, 2]
    gather = pl.kernel(
        functools.partial(_gather_body, n, epw),
        out_type=jax.ShapeDtypeStruct((ne, hh), jnp.float32),
        mesh=mesh,
        scratch_types=([pltpu.VMEM((epw,), jnp.int32),
                        pltpu.VMEM((epw,), jnp.int32),
                        pltpu.VMEM((hh,), jnp.float32),
                        pltpu.VMEM((CH,), jnp.float32)]
                       + 2 * [pltpu.VMEM((CH, hh), jnp.float32),
                              pltpu.VMEM((CH, hh), jnp.float32),
                              pltpu.VMEM((CH,), jnp.float32),
                              pltpu.VMEM((CH,), jnp.float32),
                              pltpu.VMEM((CH,), jnp.float32),
                              pltpu.VMEM((CH,), jnp.float32),
                              pltpu.VMEM((CH,), jnp.float32),
                              pltpu.VMEM((CH,), jnp.float32)]
                       + 5 * [pltpu.SemaphoreType.DMA]),
        compiler_params=pltpu.CompilerParams(use_tc_tiling_on_sc=False),
    )
    P1 = gather(A, B, x0a, x1a, x2a, We1[2 * d], es, ed)

    R = 2000
    grid = ne // R
    wm = pl.pallas_call(
        _edge_body,
        grid=(grid,),
        in_specs=[
            pl.BlockSpec((R, hh), lambda i: (i, 0)),
            pl.BlockSpec((hh, hh), lambda i: (0, 0)),
            pl.BlockSpec((1, hh), lambda i: (0, 0)),
            pl.BlockSpec((1, hh), lambda i: (0, 0)),
            pl.BlockSpec(memory_space=pltpu.MemorySpace.SMEM),
        ],
        out_specs=pl.BlockSpec((R, hh), lambda i: (i, 0)),
        out_shape=jax.ShapeDtypeStruct((ne, hh), jnp.float32),
    )(P1, We2, be2.reshape(1, hh), Wi.reshape(1, hh), bi)

    zeros = jnp.zeros((n, hh), jnp.float32)
    scatter = pl.kernel(
        functools.partial(_scatter_body, n, epw),
        out_type=jax.ShapeDtypeStruct((NC * n, hh), jnp.float32),
        mesh=mesh,
        scratch_types=[pltpu.VMEM_SHARED((n, hh), jnp.float32),
                       pltpu.VMEM((CH, hh), jnp.float32),
                       pltpu.VMEM((CH,), jnp.int32),
                       pltpu.VMEM((CH, hh), jnp.float32),
                       pltpu.VMEM((CH,), jnp.int32),
                       pltpu.SemaphoreType.DMA,
                       pltpu.SemaphoreType.DMA,
                       pltpu.SemaphoreType.DMA,
                       pltpu.SemaphoreType.DMA],
    )
    mparts = scatter(wm, es, zeros)

    h_out = pl.pallas_call(
        functools.partial(_final_body, n),
        out_shape=jax.ShapeDtypeStruct((n, d), jnp.float32),
    )(hb, mparts, Wh1[:d], Wh1[d:], bh1.reshape(1, hh), Wh2,
      bh2.reshape(1, d))

    return (h_out, e)


# speedup vs baseline: 1.3671x; 1.2449x over previous
import functools

import jax
import jax.numpy as jnp
from jax import lax
from jax.experimental import pallas as pl
from jax.experimental.pallas import tpu as pltpu
from jax.experimental.pallas import tpu_sc as plsc

NC = 2
NS = 16
NW = NC * NS
CH = 80


def _silu(v):
    return v * jax.nn.sigmoid(v)


def _prep_body(h_ref, g_ref, b_ref, wa_ref, wb_ref, be1_ref,
               hb_ref, a_ref, bb_ref):
    h = h_ref[...]
    mean = jnp.mean(h, axis=0, keepdims=True)
    var = jnp.mean((h - mean) ** 2, axis=0, keepdims=True)
    hb = g_ref[...] * (h - mean) / jnp.sqrt(var + 1e-5) + b_ref[...]
    hb_ref[...] = hb
    half_be1 = 0.5 * be1_ref[...]
    a_ref[...] = jnp.dot(hb, wa_ref[...],
                         preferred_element_type=jnp.float32) + half_be1
    bb_ref[...] = jnp.dot(hb, wb_ref[...],
                          preferred_element_type=jnp.float32) + half_be1


def _row_add_pack(dst, src, xs, xd, xsp, xdp, width):
    def blk(q, carry):
        for m in range(8):
            r = q * 8 + m
            for g in range(width // 16):
                sl = pl.ds(g * 16, 16)
                dst[r, sl] = dst[r, sl] + src[r, sl]
            xsp[q, pl.ds(m * 16, 16)] = xs[r, pl.ds(0, 16)]
            xdp[q, pl.ds(m * 16, 16)] = xd[r, pl.ds(0, 16)]
        return carry

    lax.fori_loop(0, CH // 8, blk, 0)


def _gather_body(n_nodes, epw, a_hbm, b_hbm, xp_hbm, es_hbm, ed_hbm,
                 p0_out, xs_out, xd_out,
                 ids_s, ids_d, asb0, bdb0, xsb0, xdb0, xsp0, xdp0,
                 asb1, bdb1, xsb1, xdb1, xsp1, xdp1,
                 sem_i, sem_g0, sem_g1, sem_o0, sem_o1):
    c = lax.axis_index("c")
    s = lax.axis_index("s")
    base = (s * NC + c) * epw
    cp1 = pltpu.async_copy(es_hbm.at[pl.ds(base, epw)], ids_s, sem_i)
    cp2 = pltpu.async_copy(ed_hbm.at[pl.ds(base, epw)], ids_d, sem_i)
    cp1.wait()
    cp2.wait()

    sets = ((asb0, bdb0, xsb0, xdb0, xsp0, xdp0, sem_g0, sem_o0),
            (asb1, bdb1, xsb1, xdb1, xsp1, xdp1, sem_g1, sem_o1))

    def fire(j, st):
        asb, bdb, xsb, xdb, _, _, sem_g, _ = st
        loc = j * CH
        si = ids_s.at[pl.ds(loc, CH)]
        di = ids_d.at[pl.ds(loc, CH)]
        return (pltpu.async_copy(a_hbm.at[si], asb, sem_g),
                pltpu.async_copy(b_hbm.at[di], bdb, sem_g),
                pltpu.async_copy(xp_hbm.at[si], xsb, sem_g),
                pltpu.async_copy(xp_hbm.at[di], xdb, sem_g))

    def finish(j, st, cps):
        asb, bdb, xsb, xdb, xsp, xdp, _, sem_o = st
        for cp in cps:
            cp.wait()
        _row_add_pack(asb, bdb, xsb, xdb, xsp, xdp, asb.shape[1])
        off = base + j * CH
        off8 = off // 8
        return (pltpu.async_copy(asb, p0_out.at[pl.ds(off, CH)], sem_o),
                pltpu.async_copy(xsp, xs_out.at[pl.ds(off8, CH // 8)], sem_o),
                pltpu.async_copy(xdp, xd_out.at[pl.ds(off8, CH // 8)], sem_o))

    nch = epw // CH

    def pair(jj, carry):
        j0 = jj * 2
        cps0 = fire(j0, sets[0])
        cps1 = fire(j0 + 1, sets[1])
        out0 = finish(j0, sets[0], cps0)
        out1 = finish(j0 + 1, sets[1], cps1)
        for cp in out0 + out1:
            cp.wait()
        return carry

    lax.fori_loop(0, nch // 2, pair, 0)
    if nch % 2:
        j = nch - 1
        cps = fire(j, sets[0])
        outs = finish(j, sets[0], cps)
        for cp in outs:
            cp.wait()


def _edge_body(p0_ref, xs_ref, xd_ref, w256_ref, we2_ref, be2_ref, wit_ref,
               bi_ref, out_ref):
    rr = p0_ref.shape[0]
    q = rr // 8
    diff = xs_ref[...] - xd_ref[...]
    sq = diff * diff
    jio = lax.broadcasted_iota(jnp.int32, (128, 8), 0) // 16
    mio = lax.broadcasted_iota(jnp.int32, (128, 8), 1)
    gmat = jnp.where(jio == mio, 1.0, 0.0).astype(jnp.float32)
    d2g = jnp.dot(sq, gmat, preferred_element_type=jnp.float32)
    d2rows = jnp.broadcast_to(d2g[:, None, :], (q, 8, 8)).reshape(rr, 8)
    rmod = lax.broadcasted_iota(jnp.int32, (rr, 8), 0) % 8
    cols = lax.broadcasted_iota(jnp.int32, (rr, 8), 1)
    d2col = jnp.sum(jnp.where(rmod == cols, d2rows, 0.0),
                    axis=1, keepdims=True)
    dist = jnp.sqrt(d2col)
    pre1 = p0_ref[...] + dist * w256_ref[...]
    u = _silu(pre1)
    v = jnp.dot(u, we2_ref[...], preferred_element_type=jnp.float32)
    v = _silu(v + be2_ref[...])
    logit = jnp.sum(v * wit_ref[...], axis=1, keepdims=True) + bi_ref[0]
    out_ref[...] = jax.nn.sigmoid(logit) * v


def _scatter_body(n_nodes, epw, wm_hbm, es_hbm, zeros_hbm, out_hbm,
                  acc, rows0, idx0, rows1, idx1,
                  sem_l0, sem_s0, sem_l1, sem_s1):
    c = lax.axis_index("c")
    s = lax.axis_index("s")
    npt = (n_nodes // NS) // 8 * 8
    rem = n_nodes - npt * NS
    pltpu.sync_copy(zeros_hbm.at[pl.ds(s * npt, npt)],
                    acc.at[pl.ds(s * npt, npt)])

    @pl.when(s == 0)
    def _():
        pltpu.sync_copy(zeros_hbm.at[pl.ds(npt * NS, rem)],
                        acc.at[pl.ds(npt * NS, rem)])

    plsc.subcore_barrier()
    base = (s * NC + c) * epw

    sets = ((rows0, idx0, sem_l0, sem_s0), (rows1, idx1, sem_l1, sem_s1))

    def load(j, st):
        rows, idx, sem_l, _ = st
        off = base + j * CH
        return (pltpu.async_copy(wm_hbm.at[pl.ds(off, CH)], rows, sem_l),
                pltpu.async_copy(es_hbm.at[pl.ds(off, CH)], idx, sem_l))

    def scat(st, cps):
        rows, idx, _, sem_s = st
        for cp in cps:
            cp.wait()
        return pltpu.async_copy(rows, acc.at[idx], sem_s, add=True)

    nch = epw // CH

    def pair(jj, carry):
        j0 = jj * 2
        cps0 = load(j0, sets[0])
        cps1 = load(j0 + 1, sets[1])
        sc0 = scat(sets[0], cps0)
        sc1 = scat(sets[1], cps1)
        sc0.wait()
        sc1.wait()
        return carry

    lax.fori_loop(0, nch // 2, pair, 0)
    if nch % 2:
        cps = load(nch - 1, sets[0])
        scat(sets[0], cps).wait()
    plsc.subcore_barrier()
    obase = c * n_nodes + s * npt
    pltpu.sync_copy(acc.at[pl.ds(s * npt, npt)], out_hbm.at[pl.ds(obase, npt)])

    @pl.when(s == 0)
    def _():
        pltpu.sync_copy(acc.at[pl.ds(npt * NS, rem)],
                        out_hbm.at[pl.ds(c * n_nodes + npt * NS, rem)])


def _final_body(n_nodes, hb_ref, mp_ref, wh1h_ref, wh1m_ref, bh1_ref,
                wh2_ref, bh2_ref, out_ref):
    hb = hb_ref[...]
    m = mp_ref[:n_nodes, :] + mp_ref[n_nodes:, :]
    z = (jnp.dot(hb, wh1h_ref[...], preferred_element_type=jnp.float32)
         + jnp.dot(m, wh1m_ref[...], preferred_element_type=jnp.float32)
         + bh1_ref[...])
    z = _silu(z)
    out_ref[...] = hb + jnp.dot(z, wh2_ref[...],
                                preferred_element_type=jnp.float32) + bh2_ref[...]


def kernel(h, x, e, gamma, beta, We1, be1, We2, be2, Wi, bi, Wh1, bh1, Wh2, bh2):
    n, d = h.shape
    ne = e.shape[1]
    hh = We2.shape[0]
    epw = ne // NW
    mesh = plsc.VectorSubcoreMesh(core_axis_name="c", subcore_axis_name="s")

    hb, A, B = pl.pallas_call(
        _prep_body,
        out_shape=[jax.ShapeDtypeStruct((n, d), jnp.float32),
                   jax.ShapeDtypeStruct((n, hh), jnp.float32),
                   jax.ShapeDtypeStruct((n, hh), jnp.float32)],
    )(h, gamma.reshape(1, d), beta.reshape(1, d), We1[:d], We1[d:2 * d],
      be1.reshape(1, hh))

    es = e[0]
    ed = e[1]
    xp = jnp.pad(x.astype(jnp.float32), ((0, 0), (0, 16 - x.shape[1])))
    gather = pl.kernel(
        functools.partial(_gather_body, n, epw),
        out_type=[jax.ShapeDtypeStruct((ne, hh), jnp.float32),
                  jax.ShapeDtypeStruct((ne // 8, 128), jnp.float32),
                  jax.ShapeDtypeStruct((ne // 8, 128), jnp.float32)],
        mesh=mesh,
        scratch_types=([pltpu.VMEM((epw,), jnp.int32),
                        pltpu.VMEM((epw,), jnp.int32)]
                       + 2 * [pltpu.VMEM((CH, hh), jnp.float32),
                              pltpu.VMEM((CH, hh), jnp.float32),
                              pltpu.VMEM((CH, 16), jnp.float32),
                              pltpu.VMEM((CH, 16), jnp.float32),
                              pltpu.VMEM((CH // 8, 128), jnp.float32),
                              pltpu.VMEM((CH // 8, 128), jnp.float32)]
                       + 5 * [pltpu.SemaphoreType.DMA]),
        compiler_params=pltpu.CompilerParams(use_tc_tiling_on_sc=False),
    )
    P0, Xs, Xd = gather(A, B, xp, es, ed)

    R = 1600
    grid = ne // R
    wm = pl.pallas_call(
        _edge_body,
        grid=(grid,),
        in_specs=[
            pl.BlockSpec((R, hh), lambda i: (i, 0)),
            pl.BlockSpec((R // 8, 128), lambda i: (i, 0)),
            pl.BlockSpec((R // 8, 128), lambda i: (i, 0)),
            pl.BlockSpec((1, hh), lambda i: (0, 0)),
            pl.BlockSpec((hh, hh), lambda i: (0, 0)),
            pl.BlockSpec((1, hh), lambda i: (0, 0)),
            pl.BlockSpec((1, hh), lambda i: (0, 0)),
            pl.BlockSpec(memory_space=pltpu.MemorySpace.SMEM),
        ],
        out_specs=pl.BlockSpec((R, hh), lambda i: (i, 0)),
        out_shape=jax.ShapeDtypeStruct((ne, hh), jnp.float32),
    )(P0, Xs, Xd, We1[2 * d:2 * d + 1], We2, be2.reshape(1, hh),
      Wi.reshape(1, hh), bi)

    zeros = jnp.zeros((n, hh), jnp.float32)
    scatter = pl.kernel(
        functools.partial(_scatter_body, n, epw),
        out_type=jax.ShapeDtypeStruct((NC * n, hh), jnp.float32),
        mesh=mesh,
        scratch_types=[pltpu.VMEM_SHARED((n, hh), jnp.float32),
                       pltpu.VMEM((CH, hh), jnp.float32),
                       pltpu.VMEM((CH,), jnp.int32),
                       pltpu.VMEM((CH, hh), jnp.float32),
                       pltpu.VMEM((CH,), jnp.int32),
                       pltpu.SemaphoreType.DMA,
                       pltpu.SemaphoreType.DMA,
                       pltpu.SemaphoreType.DMA,
                       pltpu.SemaphoreType.DMA],
    )
    mparts = scatter(wm, es, zeros)

    h_out = pl.pallas_call(
        functools.partial(_final_body, n),
        out_shape=jax.ShapeDtypeStruct((n, d), jnp.float32),
    )(hb, mparts, Wh1[:d], Wh1[d:], bh1.reshape(1, hh), Wh2,
      bh2.reshape(1, d))

    return (h_out, e)


# speedup vs baseline: 1.6174x; 1.1831x over previous
import functools

import jax
import jax.numpy as jnp
from jax import lax
from jax.experimental import pallas as pl
from jax.experimental.pallas import tpu as pltpu
from jax.experimental.pallas import tpu_sc as plsc

NC = 2
NS = 16
NW = NC * NS
CH = 80


def _silu(v):
    return v * jax.nn.sigmoid(v)


def _prep_body(h_ref, g_ref, b_ref, wa_ref, wb_ref, be1_ref,
               hb_ref, a_ref, bb_ref):
    h = h_ref[...]
    mean = jnp.mean(h, axis=0, keepdims=True)
    var = jnp.mean((h - mean) ** 2, axis=0, keepdims=True)
    hb = g_ref[...] * (h - mean) / jnp.sqrt(var + 1e-5) + b_ref[...]
    hb_ref[...] = hb
    half_be1 = 0.5 * be1_ref[...]
    a_ref[...] = jnp.dot(hb, wa_ref[...],
                         preferred_element_type=jnp.float32) + half_be1
    bb_ref[...] = jnp.dot(hb, wb_ref[...],
                          preferred_element_type=jnp.float32) + half_be1


def _row_add_pack(dst, src, xs, xd, xsp, xdp, width, ch):
    def blk(q, carry):
        for m in range(8):
            r = q * 8 + m
            for g in range(width // 16):
                sl = pl.ds(g * 16, 16)
                dst[r, sl] = dst[r, sl] + src[r, sl]
            xsp[q, pl.ds(m * 16, 16)] = xs[r, pl.ds(0, 16)]
            xdp[q, pl.ds(m * 16, 16)] = xd[r, pl.ds(0, 16)]
        return carry

    lax.fori_loop(0, ch // 8, blk, 0)


def _gather_body(n_nodes, epw, ch, ebase, a_hbm, b_hbm, xp_hbm, es_hbm,
                 ed_hbm,
                 p0_out, xs_out, xd_out,
                 ids_s, ids_d, asb0, bdb0, xsb0, xdb0, xsp0, xdp0,
                 asb1, bdb1, xsb1, xdb1, xsp1, xdp1,
                 sem_i, sem_g0, sem_g1, sem_o0, sem_o1):
    c = lax.axis_index("c")
    s = lax.axis_index("s")
    base = ebase + (s * NC + c) * epw
    cp1 = pltpu.async_copy(es_hbm.at[pl.ds(base, epw)], ids_s, sem_i)
    cp2 = pltpu.async_copy(ed_hbm.at[pl.ds(base, epw)], ids_d, sem_i)
    cp1.wait()
    cp2.wait()

    sets = ((asb0, bdb0, xsb0, xdb0, xsp0, xdp0, sem_g0, sem_o0),
            (asb1, bdb1, xsb1, xdb1, xsp1, xdp1, sem_g1, sem_o1))

    def fire(j, st):
        asb, bdb, xsb, xdb, _, _, sem_g, _ = st
        loc = j * ch
        si = ids_s.at[pl.ds(loc, ch)]
        di = ids_d.at[pl.ds(loc, ch)]
        return (pltpu.async_copy(a_hbm.at[si], asb, sem_g),
                pltpu.async_copy(b_hbm.at[di], bdb, sem_g),
                pltpu.async_copy(xp_hbm.at[si], xsb, sem_g),
                pltpu.async_copy(xp_hbm.at[di], xdb, sem_g))

    def finish(j, st, cps):
        asb, bdb, xsb, xdb, xsp, xdp, _, sem_o = st
        for cp in cps:
            cp.wait()
        _row_add_pack(asb, bdb, xsb, xdb, xsp, xdp, asb.shape[1], ch)
        off = base + j * ch
        off8 = off // 8
        return (pltpu.async_copy(asb, p0_out.at[pl.ds(off - ebase, ch)],
                                 sem_o),
                pltpu.async_copy(
                    xsp, xs_out.at[pl.ds(off8 - ebase // 8, ch // 8)], sem_o),
                pltpu.async_copy(
                    xdp, xd_out.at[pl.ds(off8 - ebase // 8, ch // 8)], sem_o))

    nch = epw // ch

    def pair(jj, carry):
        j0 = jj * 2
        cps0 = fire(j0, sets[0])
        cps1 = fire(j0 + 1, sets[1])
        out0 = finish(j0, sets[0], cps0)
        out1 = finish(j0 + 1, sets[1], cps1)
        for cp in out0 + out1:
            cp.wait()
        return carry

    lax.fori_loop(0, nch // 2, pair, 0)
    if nch % 2:
        j = nch - 1
        cps = fire(j, sets[0])
        outs = finish(j, sets[0], cps)
        for cp in outs:
            cp.wait()


def _edge_body(p0_ref, xs_ref, xd_ref, w256_ref, we2_ref, be2_ref, wit_ref,
               bi_ref, out_ref):
    rr = p0_ref.shape[0]
    q = rr // 8
    diff = xs_ref[...] - xd_ref[...]
    sq = diff * diff
    jio = lax.broadcasted_iota(jnp.int32, (128, 8), 0) // 16
    mio = lax.broadcasted_iota(jnp.int32, (128, 8), 1)
    gmat = jnp.where(jio == mio, 1.0, 0.0).astype(jnp.float32)
    d2g = jnp.dot(sq, gmat, preferred_element_type=jnp.float32)
    d2rows = jnp.broadcast_to(d2g[:, None, :], (q, 8, 8)).reshape(rr, 8)
    rmod = lax.broadcasted_iota(jnp.int32, (rr, 8), 0) % 8
    cols = lax.broadcasted_iota(jnp.int32, (rr, 8), 1)
    d2col = jnp.sum(jnp.where(rmod == cols, d2rows, 0.0),
                    axis=1, keepdims=True)
    dist = jnp.sqrt(d2col)
    pre1 = p0_ref[...] + dist * w256_ref[...]
    u = _silu(pre1)
    v = jnp.dot(u, we2_ref[...], preferred_element_type=jnp.float32)
    v = _silu(v + be2_ref[...])
    logit = jnp.sum(v * wit_ref[...], axis=1, keepdims=True) + bi_ref[0]
    out_ref[...] = jax.nn.sigmoid(logit) * v


def _scatter_body(n_nodes, epw, ch, ebase, wm_hbm, es_hbm, init_hbm, out_hbm,
                  acc, rows0, idx0, rows1, idx1,
                  sem_l0, sem_s0, sem_l1, sem_s1):
    c = lax.axis_index("c")
    s = lax.axis_index("s")
    npt = (n_nodes // NS) // 8 * 8
    rem = n_nodes - npt * NS
    ib = c * n_nodes
    pltpu.sync_copy(init_hbm.at[pl.ds(ib + s * npt, npt)],
                    acc.at[pl.ds(s * npt, npt)])

    @pl.when(s == 0)
    def _():
        pltpu.sync_copy(init_hbm.at[pl.ds(ib + npt * NS, rem)],
                        acc.at[pl.ds(npt * NS, rem)])

    plsc.subcore_barrier()
    base = (s * NC + c) * epw

    sets = ((rows0, idx0, sem_l0, sem_s0), (rows1, idx1, sem_l1, sem_s1))

    def load(j, st):
        rows, idx, sem_l, _ = st
        off = base + j * ch
        return (pltpu.async_copy(wm_hbm.at[pl.ds(off, ch)], rows, sem_l),
                pltpu.async_copy(es_hbm.at[pl.ds(ebase + off, ch)], idx,
                                 sem_l))

    def scat(st, cps):
        rows, idx, _, sem_s = st
        for cp in cps:
            cp.wait()
        return pltpu.async_copy(rows, acc.at[idx], sem_s, add=True)

    nch = epw // ch

    def pair(jj, carry):
        j0 = jj * 2
        cps0 = load(j0, sets[0])
        cps1 = load(j0 + 1, sets[1])
        sc0 = scat(sets[0], cps0)
        sc1 = scat(sets[1], cps1)
        sc0.wait()
        sc1.wait()
        return carry

    lax.fori_loop(0, nch // 2, pair, 0)
    if nch % 2:
        cps = load(nch - 1, sets[0])
        scat(sets[0], cps).wait()
    plsc.subcore_barrier()
    obase = c * n_nodes + s * npt
    pltpu.sync_copy(acc.at[pl.ds(s * npt, npt)], out_hbm.at[pl.ds(obase, npt)])

    @pl.when(s == 0)
    def _():
        pltpu.sync_copy(acc.at[pl.ds(npt * NS, rem)],
                        out_hbm.at[pl.ds(c * n_nodes + npt * NS, rem)])


def _final_body(n_nodes, hb_ref, mp_ref, wh1h_ref, wh1m_ref, bh1_ref,
                wh2_ref, bh2_ref, out_ref):
    hb = hb_ref[...]
    m = mp_ref[:n_nodes, :] + mp_ref[n_nodes:, :]
    z = (jnp.dot(hb, wh1h_ref[...], preferred_element_type=jnp.float32)
         + jnp.dot(m, wh1m_ref[...], preferred_element_type=jnp.float32)
         + bh1_ref[...])
    z = _silu(z)
    out_ref[...] = hb + jnp.dot(z, wh2_ref[...],
                                preferred_element_type=jnp.float32) + bh2_ref[...]


def kernel(h, x, e, gamma, beta, We1, be1, We2, be2, Wi, bi, Wh1, bh1, Wh2, bh2):
    n, d = h.shape
    ne = e.shape[1]
    hh = We2.shape[0]
    epw = ne // NW
    mesh = plsc.VectorSubcoreMesh(core_axis_name="c", subcore_axis_name="s")

    hb, A, B = pl.pallas_call(
        _prep_body,
        out_shape=[jax.ShapeDtypeStruct((n, d), jnp.float32),
                   jax.ShapeDtypeStruct((n, hh), jnp.float32),
                   jax.ShapeDtypeStruct((n, hh), jnp.float32)],
    )(h, gamma.reshape(1, d), beta.reshape(1, d), We1[:d], We1[d:2 * d],
      be1.reshape(1, hh))

    es = e[0]
    ed = e[1]
    xp = jnp.pad(x.astype(jnp.float32), ((0, 0), (0, 16 - x.shape[1])))

    NH = 2
    ne2 = ne // NH
    epw2 = ne2 // NW
    ch = 40
    R = 1600
    mparts = jnp.zeros((NC * n, hh), jnp.float32)
    for half in range(NH):
        ebase = half * ne2
        gather = pl.kernel(
            functools.partial(_gather_body, n, epw2, ch, ebase),
            out_type=[jax.ShapeDtypeStruct((ne2, hh), jnp.float32),
                      jax.ShapeDtypeStruct((ne2 // 8, 128), jnp.float32),
                      jax.ShapeDtypeStruct((ne2 // 8, 128), jnp.float32)],
            mesh=mesh,
            scratch_types=([pltpu.VMEM((epw2,), jnp.int32),
                            pltpu.VMEM((epw2,), jnp.int32)]
                           + 2 * [pltpu.VMEM((ch, hh), jnp.float32),
                                  pltpu.VMEM((ch, hh), jnp.float32),
                                  pltpu.VMEM((ch, 16), jnp.float32),
                                  pltpu.VMEM((ch, 16), jnp.float32),
                                  pltpu.VMEM((ch // 8, 128), jnp.float32),
                                  pltpu.VMEM((ch // 8, 128), jnp.float32)]
                           + 5 * [pltpu.SemaphoreType.DMA]),
            compiler_params=pltpu.CompilerParams(use_tc_tiling_on_sc=False),
        )
        P0, Xs, Xd = gather(A, B, xp, es, ed)

        wm = pl.pallas_call(
            _edge_body,
            grid=(ne2 // R,),
            in_specs=[
                pl.BlockSpec((R, hh), lambda i: (i, 0)),
                pl.BlockSpec((R // 8, 128), lambda i: (i, 0)),
                pl.BlockSpec((R // 8, 128), lambda i: (i, 0)),
                pl.BlockSpec((1, hh), lambda i: (0, 0)),
                pl.BlockSpec((hh, hh), lambda i: (0, 0)),
                pl.BlockSpec((1, hh), lambda i: (0, 0)),
                pl.BlockSpec((1, hh), lambda i: (0, 0)),
                pl.BlockSpec(memory_space=pltpu.MemorySpace.SMEM),
            ],
            out_specs=pl.BlockSpec((R, hh), lambda i: (i, 0)),
            out_shape=jax.ShapeDtypeStruct((ne2, hh), jnp.float32),
        )(P0, Xs, Xd, We1[2 * d:2 * d + 1], We2, be2.reshape(1, hh),
          Wi.reshape(1, hh), bi)

        scatter = pl.kernel(
            functools.partial(_scatter_body, n, epw2, ch, ebase),
            out_type=jax.ShapeDtypeStruct((NC * n, hh), jnp.float32),
            mesh=mesh,
            scratch_types=[pltpu.VMEM_SHARED((n, hh), jnp.float32),
                           pltpu.VMEM((ch, hh), jnp.float32),
                           pltpu.VMEM((ch,), jnp.int32),
                           pltpu.VMEM((ch, hh), jnp.float32),
                           pltpu.VMEM((ch,), jnp.int32),
                           pltpu.SemaphoreType.DMA,
                           pltpu.SemaphoreType.DMA,
                           pltpu.SemaphoreType.DMA,
                           pltpu.SemaphoreType.DMA],
        )
        mparts = scatter(wm, es, mparts)

    h_out = pl.pallas_call(
        functools.partial(_final_body, n),
        out_shape=jax.ShapeDtypeStruct((n, d), jnp.float32),
    )(hb, mparts, Wh1[:d], Wh1[d:], bh1.reshape(1, hh), Wh2,
      bh2.reshape(1, d))

    return (h_out, e)


# speedup vs baseline: 1.6189x; 1.0009x over previous
import functools

import jax
import jax.numpy as jnp
from jax import lax
from jax.experimental import pallas as pl
from jax.experimental.pallas import tpu as pltpu
from jax.experimental.pallas import tpu_sc as plsc

NC = 2
NS = 16
NW = NC * NS
CH = 80


def _silu(v):
    return v * jax.nn.sigmoid(v)


def _prep_body(h_ref, g_ref, b_ref, wa_ref, wb_ref, be1_ref,
               hb_ref, a_ref, bb_ref):
    h = h_ref[...]
    mean = jnp.mean(h, axis=0, keepdims=True)
    var = jnp.mean((h - mean) ** 2, axis=0, keepdims=True)
    hb = g_ref[...] * (h - mean) / jnp.sqrt(var + 1e-5) + b_ref[...]
    hb_ref[...] = hb
    half_be1 = 0.5 * be1_ref[...]
    a_ref[...] = jnp.dot(hb, wa_ref[...],
                         preferred_element_type=jnp.float32) + half_be1
    bb_ref[...] = jnp.dot(hb, wb_ref[...],
                          preferred_element_type=jnp.float32) + half_be1


def _row_add_pack(dst, src, xs, xd, xsp, xdp, width, ch):
    def blk(q, carry):
        for m in range(8):
            r = q * 8 + m
            for g in range(width // 16):
                sl = pl.ds(g * 16, 16)
                dst[r, sl] = dst[r, sl] + src[r, sl]
            xsp[q, pl.ds(m * 16, 16)] = xs[r, pl.ds(0, 16)]
            xdp[q, pl.ds(m * 16, 16)] = xd[r, pl.ds(0, 16)]
        return carry

    lax.fori_loop(0, ch // 8, blk, 0)


def _gather_body(n_nodes, epw, ch, ebase, a_hbm, b_hbm, xp_hbm, es_hbm,
                 ed_hbm,
                 p0_out, xs_out, xd_out,
                 ids_s, ids_d, asb0, bdb0, xsb0, xdb0, xsp0, xdp0,
                 asb1, bdb1, xsb1, xdb1, xsp1, xdp1,
                 sem_i, sem_g0, sem_g1, sem_o0, sem_o1):
    c = lax.axis_index("c")
    s = lax.axis_index("s")
    base = ebase + (s * NC + c) * epw
    cp1 = pltpu.async_copy(es_hbm.at[pl.ds(base, epw)], ids_s, sem_i)
    cp2 = pltpu.async_copy(ed_hbm.at[pl.ds(base, epw)], ids_d, sem_i)
    cp1.wait()
    cp2.wait()

    sets = ((asb0, bdb0, xsb0, xdb0, xsp0, xdp0, sem_g0, sem_o0),
            (asb1, bdb1, xsb1, xdb1, xsp1, xdp1, sem_g1, sem_o1))

    def fire(j, st):
        asb, bdb, xsb, xdb, _, _, sem_g, _ = st
        loc = j * ch
        si = ids_s.at[pl.ds(loc, ch)]
        di = ids_d.at[pl.ds(loc, ch)]
        return (pltpu.async_copy(a_hbm.at[si], asb, sem_g),
                pltpu.async_copy(b_hbm.at[di], bdb, sem_g),
                pltpu.async_copy(xp_hbm.at[si], xsb, sem_g),
                pltpu.async_copy(xp_hbm.at[di], xdb, sem_g))

    def finish(j, st, cps):
        asb, bdb, xsb, xdb, xsp, xdp, _, sem_o = st
        for cp in cps:
            cp.wait()
        _row_add_pack(asb, bdb, xsb, xdb, xsp, xdp, asb.shape[1], ch)
        off = base + j * ch
        off8 = off // 8
        return (pltpu.async_copy(asb, p0_out.at[pl.ds(off - ebase, ch)],
                                 sem_o),
                pltpu.async_copy(
                    xsp, xs_out.at[pl.ds(off8 - ebase // 8, ch // 8)], sem_o),
                pltpu.async_copy(
                    xdp, xd_out.at[pl.ds(off8 - ebase // 8, ch // 8)], sem_o))

    nch = epw // ch

    def pair(jj, carry):
        j0 = jj * 2
        cps0 = fire(j0, sets[0])
        cps1 = fire(j0 + 1, sets[1])
        out0 = finish(j0, sets[0], cps0)
        out1 = finish(j0 + 1, sets[1], cps1)
        for cp in out0 + out1:
            cp.wait()
        return carry

    lax.fori_loop(0, nch // 2, pair, 0)
    if nch % 2:
        j = nch - 1
        cps = fire(j, sets[0])
        outs = finish(j, sets[0], cps)
        for cp in outs:
            cp.wait()


def _edge_body(p0_ref, xs_ref, xd_ref, w256_ref, we2_ref, be2_ref, wit_ref,
               bi_ref, out_ref):
    rr = p0_ref.shape[0]
    q = rr // 8
    diff = xs_ref[...] - xd_ref[...]
    sq = diff * diff
    jio = lax.broadcasted_iota(jnp.int32, (128, 8), 0) // 16
    mio = lax.broadcasted_iota(jnp.int32, (128, 8), 1)
    gmat = jnp.where(jio == mio, 1.0, 0.0).astype(jnp.float32)
    d2g = jnp.dot(sq, gmat, preferred_element_type=jnp.float32)
    d2rows = jnp.broadcast_to(d2g[:, None, :], (q, 8, 8)).reshape(rr, 8)
    rmod = lax.broadcasted_iota(jnp.int32, (rr, 8), 0) % 8
    cols = lax.broadcasted_iota(jnp.int32, (rr, 8), 1)
    d2col = jnp.sum(jnp.where(rmod == cols, d2rows, 0.0),
                    axis=1, keepdims=True)
    dist = jnp.sqrt(d2col)
    pre1 = p0_ref[...] + dist * w256_ref[...]
    u = _silu(pre1)
    v = jnp.dot(u, we2_ref[...], preferred_element_type=jnp.float32)
    v = _silu(v + be2_ref[...])
    logit = jnp.sum(v * wit_ref[...], axis=1, keepdims=True) + bi_ref[0]
    out_ref[...] = jax.nn.sigmoid(logit) * v


def _scatter_body(n_nodes, epw, ch, ebase, wm_hbm, es_hbm, init_hbm, out_hbm,
                  acc, rows0, idx0, rows1, idx1,
                  sem_l0, sem_s0, sem_l1, sem_s1):
    c = lax.axis_index("c")
    s = lax.axis_index("s")
    npt = (n_nodes // NS) // 8 * 8
    rem = n_nodes - npt * NS
    ib = c * n_nodes
    pltpu.sync_copy(init_hbm.at[pl.ds(ib + s * npt, npt)],
                    acc.at[pl.ds(s * npt, npt)])

    @pl.when(s == 0)
    def _():
        pltpu.sync_copy(init_hbm.at[pl.ds(ib + npt * NS, rem)],
                        acc.at[pl.ds(npt * NS, rem)])

    plsc.subcore_barrier()
    base = (s * NC + c) * epw

    sets = ((rows0, idx0, sem_l0, sem_s0), (rows1, idx1, sem_l1, sem_s1))

    def load(j, st):
        rows, idx, sem_l, _ = st
        off = base + j * ch
        return (pltpu.async_copy(wm_hbm.at[pl.ds(off, ch)], rows, sem_l),
                pltpu.async_copy(es_hbm.at[pl.ds(ebase + off, ch)], idx,
                                 sem_l))

    def scat(st, cps):
        rows, idx, _, sem_s = st
        for cp in cps:
            cp.wait()
        return pltpu.async_copy(rows, acc.at[idx], sem_s, add=True)

    nch = epw // ch

    def pair(jj, carry):
        j0 = jj * 2
        cps0 = load(j0, sets[0])
        cps1 = load(j0 + 1, sets[1])
        sc0 = scat(sets[0], cps0)
        sc1 = scat(sets[1], cps1)
        sc0.wait()
        sc1.wait()
        return carry

    lax.fori_loop(0, nch // 2, pair, 0)
    if nch % 2:
        cps = load(nch - 1, sets[0])
        scat(sets[0], cps).wait()
    plsc.subcore_barrier()
    obase = c * n_nodes + s * npt
    pltpu.sync_copy(acc.at[pl.ds(s * npt, npt)], out_hbm.at[pl.ds(obase, npt)])

    @pl.when(s == 0)
    def _():
        pltpu.sync_copy(acc.at[pl.ds(npt * NS, rem)],
                        out_hbm.at[pl.ds(c * n_nodes + npt * NS, rem)])


def _final_body(n_nodes, hb_ref, mp_ref, wh1h_ref, wh1m_ref, bh1_ref,
                wh2_ref, bh2_ref, out_ref):
    hb = hb_ref[...]
    m = mp_ref[:n_nodes, :] + mp_ref[n_nodes:, :]
    z = (jnp.dot(hb, wh1h_ref[...], preferred_element_type=jnp.float32)
         + jnp.dot(m, wh1m_ref[...], preferred_element_type=jnp.float32)
         + bh1_ref[...])
    z = _silu(z)
    out_ref[...] = hb + jnp.dot(z, wh2_ref[...],
                                preferred_element_type=jnp.float32) + bh2_ref[...]


def kernel(h, x, e, gamma, beta, We1, be1, We2, be2, Wi, bi, Wh1, bh1, Wh2, bh2):
    n, d = h.shape
    ne = e.shape[1]
    hh = We2.shape[0]
    epw = ne // NW
    mesh = plsc.VectorSubcoreMesh(core_axis_name="c", subcore_axis_name="s")

    hb, A, B = pl.pallas_call(
        _prep_body,
        out_shape=[jax.ShapeDtypeStruct((n, d), jnp.float32),
                   jax.ShapeDtypeStruct((n, hh), jnp.float32),
                   jax.ShapeDtypeStruct((n, hh), jnp.float32)],
    )(h, gamma.reshape(1, d), beta.reshape(1, d), We1[:d], We1[d:2 * d],
      be1.reshape(1, hh))

    es = e[0]
    ed = e[1]
    xp = jnp.pad(x.astype(jnp.float32), ((0, 0), (0, 16 - x.shape[1])))

    NH = 5
    ne2 = ne // NH
    epw2 = ne2 // NW
    ch = 80
    R = 1600
    mparts = jnp.zeros((NC * n, hh), jnp.float32)
    for half in range(NH):
        ebase = half * ne2
        gather = pl.kernel(
            functools.partial(_gather_body, n, epw2, ch, ebase),
            out_type=[jax.ShapeDtypeStruct((ne2, hh), jnp.float32),
                      jax.ShapeDtypeStruct((ne2 // 8, 128), jnp.float32),
                      jax.ShapeDtypeStruct((ne2 // 8, 128), jnp.float32)],
            mesh=mesh,
            scratch_types=([pltpu.VMEM((epw2,), jnp.int32),
                            pltpu.VMEM((epw2,), jnp.int32)]
                           + 2 * [pltpu.VMEM((ch, hh), jnp.float32),
                                  pltpu.VMEM((ch, hh), jnp.float32),
                                  pltpu.VMEM((ch, 16), jnp.float32),
                                  pltpu.VMEM((ch, 16), jnp.float32),
                                  pltpu.VMEM((ch // 8, 128), jnp.float32),
                                  pltpu.VMEM((ch // 8, 128), jnp.float32)]
                           + 5 * [pltpu.SemaphoreType.DMA]),
            compiler_params=pltpu.CompilerParams(use_tc_tiling_on_sc=False),
        )
        P0, Xs, Xd = gather(A, B, xp, es, ed)

        wm = pl.pallas_call(
            _edge_body,
            grid=(ne2 // R,),
            in_specs=[
                pl.BlockSpec((R, hh), lambda i: (i, 0)),
                pl.BlockSpec((R // 8, 128), lambda i: (i, 0)),
                pl.BlockSpec((R // 8, 128), lambda i: (i, 0)),
                pl.BlockSpec((1, hh), lambda i: (0, 0)),
                pl.BlockSpec((hh, hh), lambda i: (0, 0)),
                pl.BlockSpec((1, hh), lambda i: (0, 0)),
                pl.BlockSpec((1, hh), lambda i: (0, 0)),
                pl.BlockSpec(memory_space=pltpu.MemorySpace.SMEM),
            ],
            out_specs=pl.BlockSpec((R, hh), lambda i: (i, 0)),
            out_shape=jax.ShapeDtypeStruct((ne2, hh), jnp.float32),
        )(P0, Xs, Xd, We1[2 * d:2 * d + 1], We2, be2.reshape(1, hh),
          Wi.reshape(1, hh), bi)

        scatter = pl.kernel(
            functools.partial(_scatter_body, n, epw2, ch, ebase),
            out_type=jax.ShapeDtypeStruct((NC * n, hh), jnp.float32),
            mesh=mesh,
            scratch_types=[pltpu.VMEM_SHARED((n, hh), jnp.float32),
                           pltpu.VMEM((ch, hh), jnp.float32),
                           pltpu.VMEM((ch,), jnp.int32),
                           pltpu.VMEM((ch, hh), jnp.float32),
                           pltpu.VMEM((ch,), jnp.int32),
                           pltpu.SemaphoreType.DMA,
                           pltpu.SemaphoreType.DMA,
                           pltpu.SemaphoreType.DMA,
                           pltpu.SemaphoreType.DMA],
        )
        mparts = scatter(wm, es, mparts)

    h_out = pl.pallas_call(
        functools.partial(_final_body, n),
        out_shape=jax.ShapeDtypeStruct((n, d), jnp.float32),
    )(hb, mparts, Wh1[:d], Wh1[d:], bh1.reshape(1, hh), Wh2,
      bh2.reshape(1, d))

    return (h_out, e)
